# Initial kernel scaffold; baseline (speedup 1.0000x reference)
import jax
import jax.numpy as jnp
from jax.experimental import pallas as pl


def kernel(x, weight):
    raise NotImplementedError("write your pallas kernel here")


# speedup vs baseline: 1.1092x; 1.1092x over previous
import functools

import jax
import jax.numpy as jnp
from jax import lax
from jax.experimental import pallas as pl
from jax.experimental.pallas import tpu as pltpu
from jax.experimental.pallas import tpu_sc as plsc

D = 32
IDXW = 128
K = 8
CH = K * IDXW
NC = 2
NS = 16
NW = NC * NS


def _emb_body(x_hbm, tab_hbm, out_hbm,
              idx0, idx1, rows0, rows1,
              gsem0, gsem1, osem0, osem1, *, nch):
  wid = lax.axis_index("s") * NC + lax.axis_index("c")
  wrow = wid * (nch * K)

  idx_v = (idx0, idx1)
  rows_v = (rows0, rows1)
  gsem = (gsem0, gsem1)
  osem = (osem0, osem1)

  def chunk(c, b, wait_out):
    r0 = wrow + c * K
    if wait_out:
      pltpu.make_async_copy(
          rows_v[b], out_hbm.at[pl.ds(r0 * IDXW, CH)], osem[b]).wait()
    pltpu.sync_copy(x_hbm.at[pl.ds(r0, K)], idx_v[b])
    cps = [
        pltpu.async_copy(
            tab_hbm.at[idx_v[b].at[j]],
            rows_v[b].at[pl.ds(j * IDXW, IDXW)],
            gsem[b])
        for j in range(K)
    ]
    for cp in cps:
      cp.wait()
    pltpu.async_copy(
        rows_v[b], out_hbm.at[pl.ds(r0 * IDXW, CH)], osem[b])

  chunk(0, 0, False)
  chunk(1, 1, False)

  def body(i, carry):
    chunk(2 * i, 0, True)
    chunk(2 * i + 1, 1, True)
    return carry

  npairs = nch // 2
  lax.fori_loop(1, npairs, body, 0)
  if nch % 2:
    chunk(nch - 1, 0, True)

  for b in range(2):
    pltpu.make_async_copy(
        rows_v[b], out_hbm.at[pl.ds(0, CH)], osem[b]).wait()


@functools.lru_cache(maxsize=None)
def _build(ntok):
  nch = ntok // (NW * CH)
  mesh = plsc.VectorSubcoreMesh(core_axis_name="c", subcore_axis_name="s")
  return pl.kernel(
      functools.partial(_emb_body, nch=nch),
      out_type=jax.ShapeDtypeStruct((ntok, D), jnp.float32),
      mesh=mesh,
      compiler_params=pltpu.CompilerParams(use_tc_tiling_on_sc=False),
      scratch_types=[
          pltpu.VMEM((K, IDXW), jnp.int32),
          pltpu.VMEM((K, IDXW), jnp.int32),
          pltpu.VMEM((CH, D), jnp.float32),
          pltpu.VMEM((CH, D), jnp.float32),
          pltpu.SemaphoreType.DMA,
          pltpu.SemaphoreType.DMA,
          pltpu.SemaphoreType.DMA,
          pltpu.SemaphoreType.DMA,
      ],
  )


def kernel(x, weight):
  ntok = x.shape[0] * x.shape[1]
  x2d = x.reshape(ntok // IDXW, IDXW)
  out = _build(ntok)(x2d, weight)
  return out.reshape(x.shape[0], x.shape[1], D)


# speedup vs baseline: 1.1185x; 1.0084x over previous
import functools

import jax
import jax.numpy as jnp
from jax import lax
from jax.experimental import pallas as pl
from jax.experimental.pallas import tpu as pltpu
from jax.experimental.pallas import tpu_sc as plsc

D = 32
IDXW = 128
K = 8
CH = K * IDXW
NBUF = 3
NC = 2
NS = 16
NW = NC * NS


def _emb_body(x_hbm, tab_hbm, out_hbm, idx_all,
              rows0, rows1, rows2,
              gsem0, gsem1, gsem2, osem0, osem1, osem2, *, nch):
  wid = lax.axis_index("s") * NC + lax.axis_index("c")
  wrow = wid * (nch * K)

  rows = (rows0, rows1, rows2)
  gsem = (gsem0, gsem1, gsem2)
  osem = (osem0, osem1, osem2)

  pltpu.sync_copy(x_hbm.at[pl.ds(wrow, nch * K)], idx_all)

  def fire(c, b):
    for j in range(K):
      pltpu.async_copy(
          tab_hbm.at[idx_all.at[c * K + j]],
          rows[b].at[pl.ds(j * IDXW, IDXW)],
          gsem[b])

  def refill(c, b):
    pltpu.make_async_copy(
        rows[b], out_hbm.at[pl.ds(0, CH)], osem[b]).wait()
    fire(c, b)

  def flush(c, b):
    pltpu.make_async_copy(
        tab_hbm.at[pl.ds(0, CH)], rows[b], gsem[b]).wait()
    pltpu.async_copy(
        rows[b], out_hbm.at[pl.ds((wrow + c * K) * IDXW, CH)], osem[b])

  fire(0, 0)
  fire(1, 1)
  fire(2, 2)
  flush(0, 0); refill(3, 0)
  flush(1, 1); refill(4, 1)

  def body(g, carry):
    c0 = 2 + 3 * g
    flush(c0, 2); refill(c0 + 3, 2)
    flush(c0 + 1, 0); refill(c0 + 4, 0)
    flush(c0 + 2, 1); refill(c0 + 5, 1)
    return carry

  lax.fori_loop(0, (nch - 7) // 3, body, 0)

  flush(nch - 5, 2); refill(nch - 2, 2)
  flush(nch - 4, 0); refill(nch - 1, 0)
  flush(nch - 3, 1)
  flush(nch - 2, 2)
  flush(nch - 1, 0)

  for b in range(NBUF):
    pltpu.make_async_copy(
        rows[b], out_hbm.at[pl.ds(0, CH)], osem[b]).wait()


@functools.lru_cache(maxsize=None)
def _build(ntok):
  nch = ntok // (NW * CH)
  assert nch % 3 == 1 and nch >= 7
  mesh = plsc.VectorSubcoreMesh(core_axis_name="c", subcore_axis_name="s")
  return pl.kernel(
      functools.partial(_emb_body, nch=nch),
      out_type=jax.ShapeDtypeStruct((ntok, D), jnp.float32),
      mesh=mesh,
      compiler_params=pltpu.CompilerParams(use_tc_tiling_on_sc=False),
      scratch_types=[
          pltpu.VMEM((nch * K, IDXW), jnp.int32),
          pltpu.VMEM((CH, D), jnp.float32),
          pltpu.VMEM((CH, D), jnp.float32),
          pltpu.VMEM((CH, D), jnp.float32),
          pltpu.SemaphoreType.DMA,
          pltpu.SemaphoreType.DMA,
          pltpu.SemaphoreType.DMA,
          pltpu.SemaphoreType.DMA,
          pltpu.SemaphoreType.DMA,
          pltpu.SemaphoreType.DMA,
      ],
  )


def kernel(x, weight):
  ntok = x.shape[0] * x.shape[1]
  x2d = x.reshape(ntok // IDXW, IDXW)
  out = _build(ntok)(x2d, weight)
  return out.reshape(x.shape[0], x.shape[1], D)


# speedup vs baseline: 1.2585x; 1.1252x over previous
import functools

import jax
import jax.numpy as jnp
from jax import lax
from jax.experimental import pallas as pl
from jax.experimental.pallas import tpu as pltpu
from jax.experimental.pallas import tpu_sc as plsc

F = 32
V = 1_000_000
VMAIN = V - V % 128
NPACK = V // 4
NCH1 = VMAIN // 128
NC = 2
NS = 16
NW = NC * NS
TRIP1 = 246

TOK = 16384 * 50
TPW = TOK // NW
HALF = 128
NCH2 = 200


def _iota16():
  return lax.iota(jnp.int32, 16)


def _transpose_pack(_p_c2, b, inbuf, packbuf):
  iota = _iota16()
  for q in range(32):
    for j in range(8):
      row = 16 * (j & 1) + iota
      col = jnp.full((16,), 4 * q + j // 2, jnp.int32)
      packbuf[b][q, pl.ds(16 * j, 16)] = plsc.load_gather(inbuf[b], [row, col])


def _c1_body(wt_hbm, tailp_hbm, rt_hbm,
             in0, in1, pk0, pk1, isem0, isem1, osem0, osem1):
  wid = lax.axis_index("s") * NC + lax.axis_index("c")
  inbuf = (in0, in1)
  packbuf = (pk0, pk1)
  isem = (isem0, isem1)
  osem = (osem0, osem1)

  def cidx(i):
    return jnp.minimum(wid + NW * i, NCH1 - 1)

  def fire_in(i, b):
    pltpu.async_copy(
        wt_hbm.at[pl.ds(0, F), pl.ds(cidx(i) * 128, 128)], inbuf[b], isem[b])

  @pl.when(wid == 0)
  def _():
    pltpu.sync_copy(tailp_hbm, in0.at[pl.ds(0, 16)])
    pltpu.sync_copy(in0.at[pl.ds(0, 16)], rt_hbm.at[pl.ds(NPACK - 16, 16)])

  fire_in(0, 0)
  fire_in(1, 1)

  def chunk(i, b, first):
    pltpu.make_async_copy(
        wt_hbm.at[pl.ds(0, F), pl.ds(0, 128)], inbuf[b], isem[b]).wait()
    if not first:
      pltpu.make_async_copy(
          packbuf[b], rt_hbm.at[pl.ds(0, F)], osem[b]).wait()
    _transpose_pack(None, b, inbuf, packbuf)
    fire_in(i + 2, b)
    pltpu.async_copy(
        packbuf[b], rt_hbm.at[pl.ds(cidx(i) * F, F)], osem[b])

  chunk(0, 0, True)
  chunk(1, 1, True)

  def body(j, carry):
    chunk(2 * j, 0, False)
    chunk(2 * j + 1, 1, False)
    return carry

  lax.fori_loop(1, TRIP1 // 2, body, 0)

  for b in range(2):
    pltpu.make_async_copy(
        packbuf[b], rt_hbm.at[pl.ds(0, F)], osem[b]).wait()
    pltpu.make_async_copy(
        wt_hbm.at[pl.ds(0, F), pl.ds(0, 128)], inbuf[b], isem[b]).wait()


def _c2_body(xf_hbm, rt_hbm, q_hbm,
             idxbuf, pl0, pl1, cb0, cb1, rw0, rw1, qb0, qb1,
             gsem0, gsem1, osem0, osem1):
  wid = lax.axis_index("s") * NC + lax.axis_index("c")
  tb = wid * (TPW // 50)
  plist = (pl0, pl1)
  colb = (cb0, cb1)
  rowsb = (rw0, rw1)
  qbuf = (qb0, qb1)
  gsem = (gsem0, gsem1)
  osem = (osem0, osem1)
  iota = _iota16()

  pltpu.sync_copy(xf_hbm.at[pl.ds(wid * TPW, TPW)], idxbuf)

  def fire_chunk(n, b):
    s = n // 4
    h = n % 4
    for k in range(HALF // 16):
      pos = (6400 * h + s + 800 * k) + iota * 50
      v = plsc.load_gather(idxbuf, [pos])
      plist[b][0, pl.ds(16 * k, 16)] = v >> 2
      colb[b][pl.ds(16 * k, 16)] = (v & 3) << 5
    pltpu.async_copy(
        rt_hbm.at[plist[b].at[0]], rowsb[b], gsem[b])

  fire_chunk(0, 0)
  fire_chunk(1, 1)

  def chunk(n, b, first):
    s = n // 4
    h = n % 4
    pltpu.make_async_copy(
        rt_hbm.at[pl.ds(0, HALF)], rowsb[b], gsem[b]).wait()
    if not first:
      pltpu.make_async_copy(
          qbuf[b], q_hbm.at[0, pl.ds(0, F), pl.ds(0, HALF)], osem[b]).wait()

    def extract(g, carry):
      rv = 16 * g + iota
      cb = colb[b][pl.ds(16 * g, 16)]
      for f in range(F):
        qbuf[b][f, pl.ds(16 * g, 16)] = plsc.load_gather(
            rowsb[b], [rv, cb + f])
      return carry

    lax.fori_loop(0, HALF // 16, extract, 0)

    @pl.when(n + 2 < NCH2)
    def _():
      fire_chunk(n + 2, b)

    pltpu.async_copy(
        qbuf[b], q_hbm.at[s, pl.ds(0, F), pl.ds(tb + HALF * h, HALF)],
        osem[b])

  chunk(0, 0, True)
  chunk(1, 1, True)

  def body(j, carry):
    chunk(2 * j, 0, False)
    chunk(2 * j + 1, 1, False)
    return carry

  lax.fori_loop(1, NCH2 // 2, body, 0)

  for b in range(2):
    pltpu.make_async_copy(
        qbuf[b], q_hbm.at[0, pl.ds(0, F), pl.ds(0, HALF)], osem[b]).wait()


@functools.lru_cache(maxsize=None)
def _build():
  mesh = plsc.VectorSubcoreMesh(core_axis_name="c", subcore_axis_name="s")
  params = pltpu.CompilerParams(
      use_tc_tiling_on_sc=True, needs_layout_passes=False)
  c1 = pl.kernel(
      _c1_body,
      out_type=jax.ShapeDtypeStruct((NPACK, 128), jnp.float32),
      mesh=mesh,
      compiler_params=params,
      scratch_types=[
          pltpu.VMEM((F, 128), jnp.float32),
          pltpu.VMEM((F, 128), jnp.float32),
          pltpu.VMEM((F, 128), jnp.float32),
          pltpu.VMEM((F, 128), jnp.float32),
          pltpu.SemaphoreType.DMA,
          pltpu.SemaphoreType.DMA,
          pltpu.SemaphoreType.DMA,
          pltpu.SemaphoreType.DMA,
      ],
  )
  c2 = pl.kernel(
      _c2_body,
      out_type=jax.ShapeDtypeStruct((50, F, 16384), jnp.float32),
      mesh=mesh,
      compiler_params=params,
      scratch_types=[
          pltpu.VMEM((TPW,), jnp.int32),
          pltpu.VMEM((1, 128), jnp.int32),
          pltpu.VMEM((1, 128), jnp.int32),
          pltpu.VMEM((HALF,), jnp.int32),
          pltpu.VMEM((HALF,), jnp.int32),
          pltpu.VMEM((HALF, 128), jnp.float32),
          pltpu.VMEM((HALF, 128), jnp.float32),
          pltpu.VMEM((F, HALF), jnp.float32),
          pltpu.VMEM((F, HALF), jnp.float32),
          pltpu.SemaphoreType.DMA,
          pltpu.SemaphoreType.DMA,
          pltpu.SemaphoreType.DMA,
          pltpu.SemaphoreType.DMA,
      ],
  )
  return c1, c2


_DEBUG_BYPASS_C2 = False


def kernel(x, weight):
  c1, c2 = _build()
  xf = x.reshape(-1)
  w_t = weight.T
  tailp = weight[VMAIN:, :].reshape(16, 128)
  rowtable = c1(w_t, tailp)
  if _DEBUG_BYPASS_C2:
    wrec = rowtable.reshape(V, F)
    return jnp.take(wrec, x, axis=0)
  q = c2(xf, rowtable)
  return q.transpose(2, 0, 1)
